# baseline (speedup 1.0000x reference)
import jax
import jax.numpy as jnp
from jax import lax
from jax.experimental import pallas as pl
from jax.experimental.pallas import tpu as pltpu
from jax.experimental.pallas import tpu_sc as plsc

DS = 16
D = 112
D_MIN = 2.0
D_INV_INT = 2.0
POS_MAX = 112.0 - 1e-06
SENTINEL = 100000.0

B, V, H, W = 8, 6, 256, 704
BV = B * V
HP = H // DS
WP = W // DS
NBANDS = BV * HP
NWORKERS = 32
BANDS_PER_W = NBANDS // NWORKERS
LANES = 128

GROUPS = ((0, 0), (16, 0), (28, 4))


def _min_body(x_ref, o_ref):
    x = x_ref[0, 0]
    t = jnp.where(x == 0.0, SENTINEL, x)
    r = t.reshape(HP, DS, W).min(axis=1)
    for k in (1, 2, 4, 8):
        pad = jnp.full((HP, k), SENTINEL, jnp.float32)
        r = jnp.minimum(r, jnp.concatenate([r[:, k:], pad], axis=1))
    ci = lax.broadcasted_iota(jnp.int32, (W, LANES), 0)
    ji = lax.broadcasted_iota(jnp.int32, (W, LANES), 1)
    sel = jnp.where((ci == ji * DS) & (ji < WP), 1.0, 0.0).astype(jnp.float32)
    o_ref[...] = lax.dot_general(
        r, sel, (((1,), (0,)), ((), ())),
        precision=lax.Precision.HIGHEST,
        preferred_element_type=jnp.float32)


def _sc_body(minmap, out, inmin, outbuf):
    cid = lax.axis_index("c")
    sid = lax.axis_index("s")
    wid = sid * 2 + cid

    iota = lax.iota(jnp.int32, 16)
    zeros16 = jnp.zeros((16,), jnp.float32)

    pltpu.sync_copy(minmap.at[pl.ds(wid * BANDS_PER_W, BANDS_PER_W)], inmin)

    def zrow(r, c2):
        outbuf[r, pl.ds(0, 16)] = zeros16
        outbuf[r, pl.ds(16, 16)] = zeros16
        outbuf[r, pl.ds(32, 16)] = zeros16
        return c2

    lax.fori_loop(0, D, zrow, 0)

    def band_body(i, carry):
        b = wid * BANDS_PER_W + i
        bv = b // HP
        hh = b % HP

        sites = []
        for g_base, first_lane in GROUPS:
            m = inmin[i, pl.ds(g_base, 16)]
            pos = jnp.clip((m - D_MIN) * D_INV_INT, 0.0, POS_MAX)
            lower = pos.astype(jnp.int32)
            upper = jnp.minimum(lower + 1, D - 1)
            w_upper = jnp.clip(pos - lower.astype(jnp.float32), 0.0, 1.0)
            validf = jnp.where(m < SENTINEL, 1.0, 0.0)
            w_lower = (1.0 - w_upper) * validf
            w_upper = w_upper * validf

            ww = g_base + iota
            mask = None if first_lane == 0 else (iota >= first_lane)
            plsc.addupdate_scatter(outbuf, [lower, ww], w_lower, mask=mask)
            plsc.addupdate_scatter(outbuf, [upper, ww], w_upper, mask=mask)
            sites.append((lower, upper, ww, mask))

        pltpu.sync_copy(outbuf, out.at[pl.ds(bv * D, D), hh, :])

        for lower, upper, ww, mask in sites:
            plsc.store_scatter(outbuf, [lower, ww], zeros16, mask=mask)
            plsc.store_scatter(outbuf, [upper, ww], zeros16, mask=mask)
        return carry

    lax.fori_loop(0, BANDS_PER_W, band_body, 0)


@jax.jit
def kernel(lidar_depth):
    minmap = pl.pallas_call(
        _min_body,
        grid=(BV,),
        in_specs=[pl.BlockSpec((1, 1, H, W), lambda i: (i // V, i % V, 0, 0))],
        out_specs=pl.BlockSpec((HP, LANES), lambda i: (i, 0)),
        out_shape=jax.ShapeDtypeStruct((NBANDS, LANES), jnp.float32),
    )(lidar_depth)

    mesh = plsc.VectorSubcoreMesh(core_axis_name="c", subcore_axis_name="s")
    f = pl.kernel(
        _sc_body,
        out_type=jax.ShapeDtypeStruct((BV * D, HP, LANES), jnp.float32),
        mesh=mesh,
        scratch_types=[
            pltpu.VMEM((BANDS_PER_W, LANES), jnp.float32),
            pltpu.VMEM((D, LANES), jnp.float32),
        ],
        compiler_params=pltpu.CompilerParams(
            use_tc_tiling_on_sc=False, needs_layout_passes=False
        ),
    )
    y = f(minmap)
    return y.reshape(BV, D, HP, LANES)[..., :WP]


# speedup vs baseline: 3.5276x; 3.5276x over previous
import jax
import jax.numpy as jnp
from jax import lax
from jax.experimental import pallas as pl
from jax.experimental.pallas import tpu as pltpu
from jax.experimental.pallas import tpu_sc as plsc

DS = 16
D = 112
D_MIN = 2.0
D_INV_INT = 2.0
POS_MAX = 112.0 - 1e-06
SENTINEL = 100000.0

B, V, H, W = 8, 6, 256, 704
BV = B * V
HP = H // DS
WP = W // DS
NBANDS = BV * HP
NWORKERS = 32
BANDS_PER_W = NBANDS // NWORKERS
LANES = 128

GROUPS = ((0, 0), (16, 0), (28, 4))


def _min_body(x_ref, o_ref):
    x = x_ref[0, 0]
    t = jnp.where(x == 0.0, SENTINEL, x)
    r = t.reshape(HP, DS, W).min(axis=1)
    for k in (1, 2, 4, 8):
        pad = jnp.full((HP, k), SENTINEL, jnp.float32)
        r = jnp.minimum(r, jnp.concatenate([r[:, k:], pad], axis=1))
    ci = lax.broadcasted_iota(jnp.int32, (W, LANES), 0)
    ji = lax.broadcasted_iota(jnp.int32, (W, LANES), 1)
    sel = jnp.where((ci == ji * DS) & (ji < WP), 1.0, 0.0).astype(jnp.float32)
    o_ref[...] = lax.dot_general(
        r, sel, (((1,), (0,)), ((), ())),
        precision=lax.Precision.HIGHEST,
        preferred_element_type=jnp.float32)


def _sc_body(minmap, out, inmin, outbuf):
    cid = lax.axis_index("c")
    sid = lax.axis_index("s")
    wid = sid * 2 + cid

    iota = lax.iota(jnp.int32, 16)
    zeros16 = jnp.zeros((16,), jnp.float32)

    pltpu.sync_copy(minmap.at[pl.ds(wid * BANDS_PER_W, BANDS_PER_W)], inmin)

    def zrow(r, c2):
        outbuf[r, pl.ds(0, 16)] = zeros16
        outbuf[r, pl.ds(16, 16)] = zeros16
        outbuf[r, pl.ds(32, 16)] = zeros16
        return c2

    lax.fori_loop(0, D, zrow, 0)

    def band_body(i, carry):
        b = wid * BANDS_PER_W + i
        sites = []
        for g_base, first_lane in GROUPS:
            m = inmin[i, pl.ds(g_base, 16)]
            pos = jnp.clip((m - D_MIN) * D_INV_INT, 0.0, POS_MAX)
            lower = pos.astype(jnp.int32)
            upper = jnp.minimum(lower + 1, D - 1)
            w_upper = jnp.clip(pos - lower.astype(jnp.float32), 0.0, 1.0)
            validf = jnp.where(m < SENTINEL, 1.0, 0.0)
            w_lower = (1.0 - w_upper) * validf
            w_upper = w_upper * validf

            ww = g_base + iota
            mask = None if first_lane == 0 else (iota >= first_lane)
            plsc.addupdate_scatter(outbuf, [lower, ww], w_lower, mask=mask)
            plsc.addupdate_scatter(outbuf, [upper, ww], w_upper, mask=mask)
            sites.append((lower, upper, ww, mask))

        pltpu.sync_copy(outbuf, out.at[pl.ds(b * D, D)])

        for lower, upper, ww, mask in sites:
            plsc.store_scatter(outbuf, [lower, ww], zeros16, mask=mask)
            plsc.store_scatter(outbuf, [upper, ww], zeros16, mask=mask)
        return carry

    lax.fori_loop(0, BANDS_PER_W, band_body, 0)


@jax.jit
def kernel(lidar_depth):
    minmap = pl.pallas_call(
        _min_body,
        grid=(BV,),
        in_specs=[pl.BlockSpec((1, 1, H, W), lambda i: (i // V, i % V, 0, 0))],
        out_specs=pl.BlockSpec((HP, LANES), lambda i: (i, 0)),
        out_shape=jax.ShapeDtypeStruct((NBANDS, LANES), jnp.float32),
    )(lidar_depth)

    mesh = plsc.VectorSubcoreMesh(core_axis_name="c", subcore_axis_name="s")
    f = pl.kernel(
        _sc_body,
        out_type=jax.ShapeDtypeStruct((NBANDS * D, LANES), jnp.float32),
        mesh=mesh,
        scratch_types=[
            pltpu.VMEM((BANDS_PER_W, LANES), jnp.float32),
            pltpu.VMEM((D, LANES), jnp.float32),
        ],
        compiler_params=pltpu.CompilerParams(
            use_tc_tiling_on_sc=False, needs_layout_passes=False
        ),
    )
    y = f(minmap)
    return y.reshape(BV, HP, D, LANES).transpose(0, 2, 1, 3)[..., :WP]


# speedup vs baseline: 6.6677x; 1.8902x over previous
import jax
import jax.numpy as jnp
from jax import lax
from jax.experimental import pallas as pl
from jax.experimental.pallas import tpu as pltpu
from jax.experimental.pallas import tpu_sc as plsc

DS = 16
D = 112
D_MIN = 2.0
D_INV_INT = 2.0
POS_MAX = 112.0 - 1e-06
SENTINEL = 100000.0

B, V, H, W = 8, 6, 256, 704
BV = B * V
HP = H // DS
WP = W // DS
NCOLS = BV * WP
NWORKERS = 32
COLS_PER_W = NCOLS // NWORKERS
CHUNK = 22
NCHUNKS = COLS_PER_W // CHUNK
CHUNK_ROWS = CHUNK * HP
LANES = 128


def _min_body(x_ref, o_ref):
    x = x_ref[0, 0]
    t = jnp.where(x == 0.0, SENTINEL, x)
    r = t.reshape(WP, DS, H).min(axis=1)
    for k in (1, 2, 4, 8):
        pad = jnp.full((WP, k), SENTINEL, jnp.float32)
        r = jnp.minimum(r, jnp.concatenate([r[:, k:], pad], axis=1))
    ci = lax.broadcasted_iota(jnp.int32, (H, LANES), 0)
    ji = lax.broadcasted_iota(jnp.int32, (H, LANES), 1)
    sel = jnp.where((ci == ji * DS) & (ji < HP), 1.0, 0.0).astype(jnp.float32)
    o_ref[0] = lax.dot_general(
        r, sel, (((1,), (0,)), ((), ())),
        precision=lax.Precision.HIGHEST,
        preferred_element_type=jnp.float32)


def _sc_body(minmap, out, inmin, outbuf):
    cid = lax.axis_index("c")
    sid = lax.axis_index("s")
    wid = sid * 2 + cid

    iota = lax.iota(jnp.int32, 16)
    zeros16 = jnp.zeros((16,), jnp.float32)

    pltpu.sync_copy(minmap.at[pl.ds(wid * COLS_PER_W, COLS_PER_W)], inmin)

    def zrow(r, c2):
        for j in range(LANES // 16):
            outbuf[r, pl.ds(j * 16, 16)] = zeros16
        return c2

    lax.fori_loop(0, CHUNK_ROWS, zrow, 0)

    def chunk_body(k, carry):
        sites = []
        for col in range(CHUNK):
            c_local = k * CHUNK + col
            m = inmin[c_local, pl.ds(0, 16)]
            pos = jnp.clip((m - D_MIN) * D_INV_INT, 0.0, POS_MAX)
            lower = pos.astype(jnp.int32)
            upper = jnp.minimum(lower + 1, D - 1)
            w_upper = jnp.clip(pos - lower.astype(jnp.float32), 0.0, 1.0)
            validf = jnp.where(m < SENTINEL, 1.0, 0.0)
            w_lower = (1.0 - w_upper) * validf
            w_upper = w_upper * validf

            rows = col * HP + iota
            plsc.addupdate_scatter(outbuf, [rows, lower], w_lower)
            plsc.addupdate_scatter(outbuf, [rows, upper], w_upper)
            sites.append((rows, lower, upper))

        base = (wid * COLS_PER_W + k * CHUNK) * HP
        pltpu.sync_copy(outbuf, out.at[pl.ds(base, CHUNK_ROWS)])

        for rows, lower, upper in sites:
            plsc.store_scatter(outbuf, [rows, lower], zeros16)
            plsc.store_scatter(outbuf, [rows, upper], zeros16)
        return carry

    lax.fori_loop(0, NCHUNKS, chunk_body, 0)


@jax.jit
def kernel(lidar_depth):
    xt = jnp.transpose(lidar_depth, (0, 1, 3, 2))
    minmap = pl.pallas_call(
        _min_body,
        grid=(BV,),
        in_specs=[pl.BlockSpec((1, 1, W, H), lambda i: (i // V, i % V, 0, 0))],
        out_specs=pl.BlockSpec((1, WP, LANES), lambda i: (i, 0, 0)),
        out_shape=jax.ShapeDtypeStruct((BV, WP, LANES), jnp.float32),
    )(xt)
    minmap = minmap.reshape(NCOLS, LANES)

    mesh = plsc.VectorSubcoreMesh(core_axis_name="c", subcore_axis_name="s")
    f = pl.kernel(
        _sc_body,
        out_type=jax.ShapeDtypeStruct((NCOLS * HP, LANES), jnp.float32),
        mesh=mesh,
        scratch_types=[
            pltpu.VMEM((COLS_PER_W, LANES), jnp.float32),
            pltpu.VMEM((CHUNK_ROWS, LANES), jnp.float32),
        ],
        compiler_params=pltpu.CompilerParams(
            use_tc_tiling_on_sc=False, needs_layout_passes=False
        ),
    )
    y = f(minmap)
    y = y.reshape(BV, WP, HP, LANES)[..., :D]
    return jnp.transpose(y, (0, 3, 2, 1))


# speedup vs baseline: 9.5976x; 1.4394x over previous
import jax
import jax.numpy as jnp
from jax import lax
from jax.experimental import pallas as pl
from jax.experimental.pallas import tpu as pltpu
from jax.experimental.pallas import tpu_sc as plsc

DS = 16
D = 112
D_MIN = 2.0
D_INV_INT = 2.0
POS_MAX = 112.0 - 1e-06
SENTINEL = 100000.0

B, V, H, W = 8, 6, 256, 704
BV = B * V
HP = H // DS
WP = W // DS
NCOLS = BV * WP
NWORKERS = 32
COLS_PER_W = NCOLS // NWORKERS
CHUNK = 22
NCHUNKS = COLS_PER_W // CHUNK
CHUNK_ROWS = CHUNK * HP
LANES = 128


def _min_body(sel_ref, x_ref, o_ref):
    x = x_ref[0].reshape(V * W, H)
    t = jnp.where(x == 0.0, SENTINEL, x)
    rows = []
    for g in range(V * WP):
        blk = t[g * DS:(g + 1) * DS]
        m8 = jnp.minimum(blk[0:8], blk[8:16])
        m4 = jnp.minimum(m8[0:4], m8[4:8])
        m2 = jnp.minimum(m4[0:2], m4[2:4])
        rows.append(jnp.minimum(m2[0:1], m2[1:2]))
    r = jnp.concatenate(rows, axis=0)
    for k in (1, 2, 4, 8):
        pad = jnp.full((V * WP, k), SENTINEL, jnp.float32)
        r = jnp.minimum(r, jnp.concatenate([r[:, k:], pad], axis=1))
    sel = sel_ref[...]
    hi = r.astype(jnp.bfloat16)
    rem = r - hi.astype(jnp.float32)
    mid = rem.astype(jnp.bfloat16)
    lo = (rem - mid.astype(jnp.float32)).astype(jnp.bfloat16)
    dn = (((1,), (0,)), ((), ()))
    acc = lax.dot_general(hi, sel, dn, preferred_element_type=jnp.float32)
    acc = acc + lax.dot_general(mid, sel, dn,
                                preferred_element_type=jnp.float32)
    acc = acc + lax.dot_general(lo, sel, dn,
                                preferred_element_type=jnp.float32)
    o_ref[...] = acc.reshape(V, WP, LANES)


def _sc_body(minmap, out, inmin, outbuf):
    cid = lax.axis_index("c")
    sid = lax.axis_index("s")
    wid = sid * 2 + cid

    iota = lax.iota(jnp.int32, 16)
    zeros16 = jnp.zeros((16,), jnp.float32)

    pltpu.sync_copy(minmap.at[pl.ds(wid * COLS_PER_W, COLS_PER_W)], inmin)

    def zrow(r, c2):
        for j in range(LANES // 16):
            outbuf[r, pl.ds(j * 16, 16)] = zeros16
        return c2

    lax.fori_loop(0, CHUNK_ROWS, zrow, 0)

    def chunk_body(k, carry):
        sites = []
        for col in range(CHUNK):
            c_local = k * CHUNK + col
            m = inmin[c_local, pl.ds(0, 16)]
            pos = jnp.clip((m - D_MIN) * D_INV_INT, 0.0, POS_MAX)
            lower = pos.astype(jnp.int32)
            upper = jnp.minimum(lower + 1, D - 1)
            w_upper = jnp.clip(pos - lower.astype(jnp.float32), 0.0, 1.0)
            validf = jnp.where(m < SENTINEL, 1.0, 0.0)
            w_lower = (1.0 - w_upper) * validf
            w_upper = w_upper * validf

            rows = col * HP + iota
            plsc.addupdate_scatter(outbuf, [rows, lower], w_lower)
            plsc.addupdate_scatter(outbuf, [rows, upper], w_upper)
            sites.append((rows, lower, upper))

        base = (wid * COLS_PER_W + k * CHUNK) * HP
        pltpu.sync_copy(outbuf, out.at[pl.ds(base, CHUNK_ROWS)])

        for rows, lower, upper in sites:
            plsc.store_scatter(outbuf, [rows, lower], zeros16)
            plsc.store_scatter(outbuf, [rows, upper], zeros16)
        return carry

    lax.fori_loop(0, NCHUNKS, chunk_body, 0)


@jax.jit
def kernel(lidar_depth):
    xt = jnp.transpose(lidar_depth, (0, 1, 3, 2))
    ci = lax.broadcasted_iota(jnp.int32, (H, LANES), 0)
    ji = lax.broadcasted_iota(jnp.int32, (H, LANES), 1)
    sel = jnp.where((ci == ji * DS) & (ji < HP), 1.0, 0.0).astype(jnp.bfloat16)
    minmap = pl.pallas_call(
        _min_body,
        grid=(B,),
        in_specs=[
            pl.BlockSpec((H, LANES), lambda i: (0, 0)),
            pl.BlockSpec((1, V, W, H), lambda i: (i, 0, 0, 0)),
        ],
        out_specs=pl.BlockSpec((V, WP, LANES), lambda i: (i, 0, 0)),
        out_shape=jax.ShapeDtypeStruct((BV, WP, LANES), jnp.float32),
    )(sel, xt)
    minmap = minmap.reshape(NCOLS, LANES)

    mesh = plsc.VectorSubcoreMesh(core_axis_name="c", subcore_axis_name="s")
    f = pl.kernel(
        _sc_body,
        out_type=jax.ShapeDtypeStruct((NCOLS * HP, LANES), jnp.float32),
        mesh=mesh,
        scratch_types=[
            pltpu.VMEM((COLS_PER_W, LANES), jnp.float32),
            pltpu.VMEM((CHUNK_ROWS, LANES), jnp.float32),
        ],
        compiler_params=pltpu.CompilerParams(
            use_tc_tiling_on_sc=False, needs_layout_passes=False
        ),
    )
    y = f(minmap)
    y = y.reshape(BV, WP, HP, LANES)[..., :D]
    return jnp.transpose(y, (0, 3, 2, 1))


# speedup vs baseline: 10.0238x; 1.0444x over previous
import jax
import jax.numpy as jnp
from jax import lax
from jax.experimental import pallas as pl
from jax.experimental.pallas import tpu as pltpu
from jax.experimental.pallas import tpu_sc as plsc

DS = 16
D = 112
D_MIN = 2.0
D_INV_INT = 2.0
POS_MAX = 112.0 - 1e-06
SENTINEL = 100000.0

B, V, H, W = 8, 6, 256, 704
BV = B * V
HP = H // DS
WP = W // DS
NCOLS = BV * WP
NWORKERS = 32
COLS_PER_W = NCOLS // NWORKERS
CHUNK = 22
NCHUNKS = COLS_PER_W // CHUNK
CHUNK_ROWS = CHUNK * HP
LANES = 128


def _min_body(sel_ref, x_ref, o_ref):
    x = x_ref[0].reshape(V * W, H)
    t = jnp.where(x == 0.0, SENTINEL, x)
    rows = []
    for g in range(V * WP):
        blk = t[g * DS:(g + 1) * DS]
        m8 = jnp.minimum(blk[0:8], blk[8:16])
        m4 = jnp.minimum(m8[0:4], m8[4:8])
        m2 = jnp.minimum(m4[0:2], m4[2:4])
        rows.append(jnp.minimum(m2[0:1], m2[1:2]))
    r = jnp.concatenate(rows, axis=0)
    for k in (1, 2, 4, 8):
        pad = jnp.full((V * WP, k), SENTINEL, jnp.float32)
        r = jnp.minimum(r, jnp.concatenate([r[:, k:], pad], axis=1))
    sel = sel_ref[...]
    hi = r.astype(jnp.bfloat16)
    rem = r - hi.astype(jnp.float32)
    mid = rem.astype(jnp.bfloat16)
    lo = (rem - mid.astype(jnp.float32)).astype(jnp.bfloat16)
    dn = (((1,), (0,)), ((), ()))
    acc = lax.dot_general(hi, sel, dn, preferred_element_type=jnp.float32)
    acc = acc + lax.dot_general(mid, sel, dn,
                                preferred_element_type=jnp.float32)
    acc = acc + lax.dot_general(lo, sel, dn,
                                preferred_element_type=jnp.float32)
    o_ref[...] = acc


def _sc_body(minmap, out, inmin, outbuf):
    cid = lax.axis_index("c")
    sid = lax.axis_index("s")
    wid = sid * 2 + cid

    iota = lax.iota(jnp.int32, 16)
    zeros16 = jnp.zeros((16,), jnp.float32)

    pltpu.sync_copy(minmap.at[pl.ds(wid * COLS_PER_W, COLS_PER_W)], inmin)

    def zrow(r, c2):
        for j in range(LANES // 16):
            outbuf[r, pl.ds(j * 16, 16)] = zeros16
        return c2

    lax.fori_loop(0, CHUNK_ROWS, zrow, 0)

    def chunk_body(k, carry):
        sites = []
        for col in range(CHUNK):
            c_local = k * CHUNK + col
            m = inmin[c_local, pl.ds(0, 16)]
            pos = jnp.clip((m - D_MIN) * D_INV_INT, 0.0, POS_MAX)
            lower = pos.astype(jnp.int32)
            upper = jnp.minimum(lower + 1, D - 1)
            w_upper = jnp.clip(pos - lower.astype(jnp.float32), 0.0, 1.0)
            validf = jnp.where(m < SENTINEL, 1.0, 0.0)
            w_lower = (1.0 - w_upper) * validf
            w_upper = w_upper * validf

            rows = col * HP + iota
            plsc.addupdate_scatter(outbuf, [rows, lower], w_lower)
            plsc.addupdate_scatter(outbuf, [rows, upper], w_upper)
            sites.append((rows, lower, upper))

        base = (wid * COLS_PER_W + k * CHUNK) * HP
        pltpu.sync_copy(outbuf, out.at[pl.ds(base, CHUNK_ROWS)])

        for rows, lower, upper in sites:
            plsc.store_scatter(outbuf, [rows, lower], zeros16)
            plsc.store_scatter(outbuf, [rows, upper], zeros16)
        return carry

    lax.fori_loop(0, NCHUNKS, chunk_body, 0)


@jax.jit
def kernel(lidar_depth):
    xt = jnp.transpose(lidar_depth, (0, 1, 3, 2))
    ci = lax.broadcasted_iota(jnp.int32, (H, LANES), 0)
    ji = lax.broadcasted_iota(jnp.int32, (H, LANES), 1)
    sel = jnp.where((ci == ji * DS) & (ji < HP), 1.0, 0.0).astype(jnp.bfloat16)
    minmap = pl.pallas_call(
        _min_body,
        grid=(B,),
        in_specs=[
            pl.BlockSpec((H, LANES), lambda i: (0, 0)),
            pl.BlockSpec((1, V, W, H), lambda i: (i, 0, 0, 0)),
        ],
        out_specs=pl.BlockSpec((V * WP, LANES), lambda i: (i, 0)),
        out_shape=jax.ShapeDtypeStruct((NCOLS, LANES), jnp.float32),
    )(sel, xt)

    mesh = plsc.VectorSubcoreMesh(core_axis_name="c", subcore_axis_name="s")
    f = pl.kernel(
        _sc_body,
        out_type=jax.ShapeDtypeStruct((NCOLS * HP, LANES), jnp.float32),
        mesh=mesh,
        scratch_types=[
            pltpu.VMEM((COLS_PER_W, LANES), jnp.float32),
            pltpu.VMEM((CHUNK_ROWS, LANES), jnp.float32),
        ],
        compiler_params=pltpu.CompilerParams(
            use_tc_tiling_on_sc=False, needs_layout_passes=False
        ),
    )
    y = f(minmap)
    y = y.reshape(BV, WP, HP, LANES)[..., :D]
    return jnp.transpose(y, (0, 3, 2, 1))


# speedup vs baseline: 10.3300x; 1.0305x over previous
import jax
import jax.numpy as jnp
from jax import lax
from jax.experimental import pallas as pl
from jax.experimental.pallas import tpu as pltpu
from jax.experimental.pallas import tpu_sc as plsc

DS = 16
D = 112
D_MIN = 2.0
D_INV_INT = 2.0
POS_MAX = 112.0 - 1e-06
SENTINEL = 100000.0

B, V, H, W = 8, 6, 256, 704
BV = B * V
HP = H // DS
WP = W // DS
NCOLS = BV * WP
NWORKERS = 32
COLS_PER_W = NCOLS // NWORKERS
CHUNK = 22
NCHUNKS = COLS_PER_W // CHUNK
CHUNK_ROWS = CHUNK * HP
LANES = 128


def _min_body(sel_ref, x_ref, o_ref):
    x = x_ref[...].reshape(2 * V * W, H)
    t = jnp.where(x == 0.0, SENTINEL, x)
    rows = []
    for g in range(2 * V * WP):
        blk = t[g * DS:(g + 1) * DS]
        m8 = jnp.minimum(blk[0:8], blk[8:16])
        m4 = jnp.minimum(m8[0:4], m8[4:8])
        m2 = jnp.minimum(m4[0:2], m4[2:4])
        rows.append(jnp.minimum(m2[0:1], m2[1:2]))
    r = jnp.concatenate(rows, axis=0)
    for k in (1, 2, 4, 8):
        pad = jnp.full((2 * V * WP, k), SENTINEL, jnp.float32)
        r = jnp.minimum(r, jnp.concatenate([r[:, k:], pad], axis=1))
    sel = sel_ref[...]
    hi = r.astype(jnp.bfloat16)
    rem = r - hi.astype(jnp.float32)
    mid = rem.astype(jnp.bfloat16)
    lo = (rem - mid.astype(jnp.float32)).astype(jnp.bfloat16)
    dn = (((1,), (0,)), ((), ()))
    acc = lax.dot_general(hi, sel, dn, preferred_element_type=jnp.float32)
    acc = acc + lax.dot_general(mid, sel, dn,
                                preferred_element_type=jnp.float32)
    acc = acc + lax.dot_general(lo, sel, dn,
                                preferred_element_type=jnp.float32)
    o_ref[...] = acc


def _sc_body(minmap, out, inmin, outbuf):
    cid = lax.axis_index("c")
    sid = lax.axis_index("s")
    wid = sid * 2 + cid

    iota = lax.iota(jnp.int32, 16)
    zeros16 = jnp.zeros((16,), jnp.float32)

    pltpu.sync_copy(minmap.at[pl.ds(wid * COLS_PER_W, COLS_PER_W)], inmin)

    def zrow(r, c2):
        for j in range(D // 16):
            outbuf[r, pl.ds(j * 16, 16)] = zeros16
        return c2

    lax.fori_loop(0, CHUNK_ROWS, zrow, 0)

    def chunk_body(k, carry):
        sites = []
        for col in range(CHUNK):
            c_local = k * CHUNK + col
            m = inmin[c_local, pl.ds(0, 16)]
            pos = jnp.clip((m - D_MIN) * D_INV_INT, 0.0, POS_MAX)
            lower = pos.astype(jnp.int32)
            upper = jnp.minimum(lower + 1, D - 1)
            w_upper = jnp.clip(pos - lower.astype(jnp.float32), 0.0, 1.0)
            validf = jnp.where(m < SENTINEL, 1.0, 0.0)
            w_lower = (1.0 - w_upper) * validf
            w_upper = w_upper * validf

            rows = col * HP + iota
            plsc.addupdate_scatter(outbuf, [rows, lower], w_lower)
            plsc.addupdate_scatter(outbuf, [rows, upper], w_upper)
            sites.append((rows, lower, upper))

        base = (wid * COLS_PER_W + k * CHUNK) * HP
        pltpu.sync_copy(outbuf, out.at[pl.ds(base, CHUNK_ROWS)])

        for rows, lower, upper in sites:
            plsc.store_scatter(outbuf, [rows, lower], zeros16)
            plsc.store_scatter(outbuf, [rows, upper], zeros16)
        return carry

    lax.fori_loop(0, NCHUNKS, chunk_body, 0)


@jax.jit
def kernel(lidar_depth):
    xt = jnp.transpose(lidar_depth, (0, 1, 3, 2))
    ci = lax.broadcasted_iota(jnp.int32, (H, LANES), 0)
    ji = lax.broadcasted_iota(jnp.int32, (H, LANES), 1)
    sel = jnp.where((ci == ji * DS) & (ji < HP), 1.0, 0.0).astype(jnp.bfloat16)
    minmap = pl.pallas_call(
        _min_body,
        grid=(B // 2,),
        in_specs=[
            pl.BlockSpec((H, LANES), lambda i: (0, 0)),
            pl.BlockSpec((2, V, W, H), lambda i: (i, 0, 0, 0)),
        ],
        out_specs=pl.BlockSpec((2 * V * WP, LANES), lambda i: (i, 0)),
        out_shape=jax.ShapeDtypeStruct((NCOLS, LANES), jnp.float32),
    )(sel, xt)

    mesh = plsc.VectorSubcoreMesh(core_axis_name="c", subcore_axis_name="s")
    f = pl.kernel(
        _sc_body,
        out_type=jax.ShapeDtypeStruct((NCOLS * HP, LANES), jnp.float32),
        mesh=mesh,
        scratch_types=[
            pltpu.VMEM((COLS_PER_W, LANES), jnp.float32),
            pltpu.VMEM((CHUNK_ROWS, LANES), jnp.float32),
        ],
        compiler_params=pltpu.CompilerParams(
            use_tc_tiling_on_sc=False, needs_layout_passes=False
        ),
    )
    y = f(minmap)
    y = y.reshape(BV, WP, HP, LANES)[..., :D]
    return jnp.transpose(y, (0, 3, 2, 1))
